# baseline (speedup 1.0000x reference)
import functools

import jax
import jax.numpy as jnp
from jax import lax
from jax.experimental import pallas as pl
from jax.experimental.pallas import tpu as pltpu
from jax.experimental.pallas import tpu_sc as plsc

D = 128
SEQ = 200
HALF = 100
NLANE = 16
NREG = D // NLANE
NC = 2
NS = 16
NW = NC * NS
EPS = 1e-5
UNROLL = 5


def _body(x_hbm, seg_hbm, tok_hbm, pos_hbm, segtab_hbm, gamma_hbm, beta_hbm,
          out_hbm, idx_v, rows_v, out_v, pos_v, segtab_v, gamma_v, beta_v,
          seg_v, sem):
    wid = lax.axis_index("s") * NC + lax.axis_index("c")
    nbatch = x_hbm.shape[0]
    seqs_per_w = nbatch // NW

    pltpu.sync_copy(pos_hbm.at[pl.ds(0, SEQ)], pos_v)
    pltpu.sync_copy(segtab_hbm, segtab_v)
    pltpu.sync_copy(gamma_hbm, gamma_v)
    pltpu.sync_copy(beta_hbm, beta_v)

    def seq_body(s, carry):
        row = wid * seqs_per_w + s
        pltpu.sync_copy(x_hbm.at[row], idx_v)
        pltpu.sync_copy(seg_hbm.at[row], seg_v)
        cp0 = pltpu.async_copy(tok_hbm.at[idx_v.at[0]],
                               rows_v.at[pl.ds(0, HALF)], sem)
        cp1 = pltpu.async_copy(tok_hbm.at[idx_v.at[1]],
                               rows_v.at[pl.ds(HALF, HALF)], sem)
        cp0.wait()
        cp1.wait()

        def process(p, sgi):
            vs = []
            for k in range(NREG):
                sl = pl.ds(k * NLANE, NLANE)
                v = rows_v[p, sl] + pos_v[p, sl] + segtab_v[sgi, sl]
                vs.append(v)
            tot = ((vs[0] + vs[1]) + (vs[2] + vs[3])) + \
                  ((vs[4] + vs[5]) + (vs[6] + vs[7]))
            sq = [v * v for v in vs]
            tot2 = ((sq[0] + sq[1]) + (sq[2] + sq[3])) + \
                   ((sq[4] + sq[5]) + (sq[6] + sq[7]))
            s1 = jnp.sum(tot)
            s2 = jnp.sum(tot2)
            mean = s1 * (1.0 / D)
            var = s2 * (1.0 / D) - mean * mean + EPS
            xh = 0.5 * var
            ii = lax.bitcast_convert_type(var, jnp.int32)
            ii = 0x5F3759DF - lax.shift_right_logical(ii, 1)
            y = lax.bitcast_convert_type(ii, jnp.float32)
            y = y * (1.5 - xh * y * y)
            y = y * (1.5 - xh * y * y)
            y = y * (1.5 - xh * y * y)
            for k in range(NREG):
                sl = pl.ds(k * NLANE, NLANE)
                out_v[p, sl] = ((vs[k] - mean) * y) * gamma_v[sl] \
                    + beta_v[sl]

        for j in range(2):
            def group_body(g, c2, j=j):
                base = g * UNROLL
                segv = seg_v[j, pl.ds(base, NLANE)]
                for l in range(UNROLL):
                    process(j * HALF + base + l, segv[l])
                return c2
            lax.fori_loop(0, HALF // UNROLL, group_body, 0)

        pltpu.sync_copy(out_v, out_hbm.at[pl.ds(row * SEQ, SEQ)])
        return carry

    lax.fori_loop(0, seqs_per_w, seq_body, 0)


def kernel(x, seg, tok_embed, pos_embed, seg_embed, gamma, beta):
    b, s = x.shape
    x3 = x.reshape(b, 2, s // 2).astype(jnp.int32)
    seg3 = jnp.pad(seg.reshape(b, 2, s // 2).astype(jnp.int32),
                   ((0, 0), (0, 0), (0, D - s // 2)))

    run = functools.partial(
        pl.kernel,
        out_type=jax.ShapeDtypeStruct((b * s, D), jnp.float32),
        scratch_types=[
            pltpu.VMEM((2, HALF), jnp.int32),
            pltpu.VMEM((SEQ, D), jnp.float32),
            pltpu.VMEM((SEQ, D), jnp.float32),
            pltpu.VMEM((SEQ, D), jnp.float32),
            pltpu.VMEM((2, D), jnp.float32),
            pltpu.VMEM((D,), jnp.float32),
            pltpu.VMEM((D,), jnp.float32),
            pltpu.VMEM((2, D), jnp.int32),
            pltpu.SemaphoreType.DMA,
        ],
        mesh=plsc.VectorSubcoreMesh(core_axis_name="c", subcore_axis_name="s"),
        compiler_params=pltpu.CompilerParams(needs_layout_passes=False),
    )(_body)

    out = run(x3, seg3, tok_embed, pos_embed, seg_embed, gamma, beta)
    return out.reshape(b, s, D)


# speedup vs baseline: 1.8831x; 1.8831x over previous
import functools

import jax
import jax.numpy as jnp
from jax import lax
from jax.experimental import pallas as pl
from jax.experimental.pallas import tpu as pltpu
from jax.experimental.pallas import tpu_sc as plsc

D = 128
SEQ = 200
HALF = 100
NLANE = 16
NREG = D // NLANE
NC = 2
NS = 16
NW = NC * NS
EPS = 1e-5
UNROLL = 4


def _body(x_hbm, seg_hbm, tok_hbm, pos_hbm, segtab_hbm, gamma_hbm, beta_hbm,
          out_hbm, idx_v, rows_v, out_v, pos_v, segtab_v, gamma_v, beta_v,
          seg_v, sem):
    wid = lax.axis_index("s") * NC + lax.axis_index("c")
    nbatch = x_hbm.shape[0]
    seqs_per_w = nbatch // NW

    pltpu.sync_copy(pos_hbm.at[pl.ds(0, SEQ)], pos_v)
    pltpu.sync_copy(segtab_hbm, segtab_v)
    pltpu.sync_copy(gamma_hbm, gamma_v)
    pltpu.sync_copy(beta_hbm, beta_v)

    def seq_body(s, carry):
        row = wid * seqs_per_w + s
        pltpu.sync_copy(x_hbm.at[row], idx_v)
        pltpu.sync_copy(seg_hbm.at[row], seg_v)
        cp0 = pltpu.async_copy(tok_hbm.at[idx_v.at[0]],
                               rows_v.at[pl.ds(0, HALF)], sem)
        cp1 = pltpu.async_copy(tok_hbm.at[idx_v.at[1]],
                               rows_v.at[pl.ds(HALF, HALF)], sem)
        cp0.wait()
        cp1.wait()

        def process(p, sgi):
            vs = []
            for k in range(NREG):
                sl = pl.ds(k * NLANE, NLANE)
                v = rows_v[p, sl] + pos_v[p, sl] + segtab_v[sgi, sl]
                vs.append(v)
            tot = ((vs[0] + vs[1]) + (vs[2] + vs[3])) + \
                  ((vs[4] + vs[5]) + (vs[6] + vs[7]))
            sq = [v * v for v in vs]
            tot2 = ((sq[0] + sq[1]) + (sq[2] + sq[3])) + \
                   ((sq[4] + sq[5]) + (sq[6] + sq[7]))
            s1 = jnp.sum(tot)
            s2 = jnp.sum(tot2)
            mean = s1 * (1.0 / D)
            var = s2 * (1.0 / D) - mean * mean + EPS
            xh = 0.5 * var
            ii = lax.bitcast_convert_type(var, jnp.int32)
            ii = 0x5F3759DF - lax.shift_right_logical(ii, 1)
            y = lax.bitcast_convert_type(ii, jnp.float32)
            y = y * (1.5 - xh * y * y)
            y = y * (1.5 - xh * y * y)
            y = y * (1.5 - xh * y * y)
            for k in range(NREG):
                sl = pl.ds(k * NLANE, NLANE)
                out_v[p, sl] = ((vs[k] - mean) * y) * gamma_v[sl] \
                    + beta_v[sl]

        for j in range(2):
            @plsc.parallel_loop(0, HALF, 1, unroll=UNROLL)
            def tok_loop(i, j=j):
                segv = seg_v[j, pl.ds(i, NLANE)]
                process(j * HALF + i, segv[0])

        pltpu.sync_copy(out_v, out_hbm.at[pl.ds(row * SEQ, SEQ)])
        return carry

    lax.fori_loop(0, seqs_per_w, seq_body, 0)


def kernel(x, seg, tok_embed, pos_embed, seg_embed, gamma, beta):
    b, s = x.shape
    x3 = x.reshape(b, 2, s // 2).astype(jnp.int32)
    seg3 = jnp.pad(seg.reshape(b, 2, s // 2).astype(jnp.int32),
                   ((0, 0), (0, 0), (0, D - s // 2)))

    run = functools.partial(
        pl.kernel,
        out_type=jax.ShapeDtypeStruct((b * s, D), jnp.float32),
        scratch_types=[
            pltpu.VMEM((2, HALF), jnp.int32),
            pltpu.VMEM((SEQ, D), jnp.float32),
            pltpu.VMEM((SEQ, D), jnp.float32),
            pltpu.VMEM((SEQ, D), jnp.float32),
            pltpu.VMEM((2, D), jnp.float32),
            pltpu.VMEM((D,), jnp.float32),
            pltpu.VMEM((D,), jnp.float32),
            pltpu.VMEM((2, D), jnp.int32),
            pltpu.SemaphoreType.DMA,
        ],
        mesh=plsc.VectorSubcoreMesh(core_axis_name="c", subcore_axis_name="s"),
        compiler_params=pltpu.CompilerParams(needs_layout_passes=False),
    )(_body)

    out = run(x3, seg3, tok_embed, pos_embed, seg_embed, gamma, beta)
    return out.reshape(b, s, D)
